# baseline (speedup 1.0000x reference)
import functools
import jax
import jax.numpy as jnp
from jax.experimental import pallas as pl
from jax.experimental.pallas import tpu as pltpu

_D = 128
_N = 10000
_E = 320000
_EPS = 1e-5
_EBLK = 512
_NBLK = 1000


def _edge_kernel(a_ref, b_ref, ve_ref, W1_ref, b1_ref, W2p_ref, b2p_ref,
                 We1_ref, We2_ref, be_ref, f_ref, vepre_ref, stats_ref):
    x = jnp.concatenate([a_ref[...], b_ref[...], ve_ref[...]], axis=1)
    h = jnp.maximum(x @ W1_ref[...] + b1_ref[...], 0.0)
    m = h @ W2p_ref[...] + b2p_ref[...]
    k = jax.nn.sigmoid(m[:, 4 * _D:4 * _D + 1])
    f = m[:, :4 * _D] * k
    f_ref[...] = f
    vp = f @ We1_ref[...] + ve_ref[...] @ We2_ref[...] + be_ref[...]
    vepre_ref[...] = vp

    @pl.when(pl.program_id(0) == 0)
    def _():
        stats_ref[...] = jnp.zeros_like(stats_ref)

    s = jnp.sum(vp, axis=0, keepdims=True)
    s2 = jnp.sum(vp * vp, axis=0, keepdims=True)
    stats_ref[0:1, :] += s
    stats_ref[1:2, :] += s2


def _node_kernel(vc_ref, nf1_ref, nf2_ref, nf3_ref, nf4_ref, Wr_ref, br_ref,
                 out_ref, stats_ref):
    Wr = Wr_ref[...]
    o = (vc_ref[...] @ Wr[0:_D] + nf1_ref[...] @ Wr[_D:2 * _D]
         + nf2_ref[...] @ Wr[2 * _D:3 * _D] + nf3_ref[...] @ Wr[3 * _D:4 * _D]
         + nf4_ref[...] @ Wr[4 * _D:5 * _D] + br_ref[...])
    out_ref[...] = o

    @pl.when(pl.program_id(0) == 0)
    def _():
        stats_ref[...] = jnp.zeros_like(stats_ref)

    stats_ref[0:1, :] += jnp.sum(o, axis=0, keepdims=True)
    stats_ref[1:2, :] += jnp.sum(o * o, axis=0, keepdims=True)


def _norm_kernel(x_ref, scale_ref, shift_ref, o_ref):
    o_ref[...] = x_ref[...] * scale_ref[...] + shift_ref[...]


def _normalize(x, scale, shift, blk):
    n = x.shape[0]
    return pl.pallas_call(
        _norm_kernel,
        grid=(n // blk,),
        in_specs=[
            pl.BlockSpec((blk, _D), lambda i: (i, 0)),
            pl.BlockSpec((1, _D), lambda i: (0, 0)),
            pl.BlockSpec((1, _D), lambda i: (0, 0)),
        ],
        out_specs=pl.BlockSpec((blk, _D), lambda i: (i, 0)),
        out_shape=jax.ShapeDtypeStruct((n, _D), jnp.float32),
    )(x, scale, shift)


def kernel(in_vc, in_ve, edge_index, W1, b1, W2, b2, Wr, br, We, be,
           gamma_gc, beta_gc, gamma_ef, beta_ef):
    src = edge_index[0]
    dst = edge_index[1]

    W2p = jnp.concatenate(
        [W2[:, 1:], W2[:, 0:1], jnp.zeros((3 * _D, _D - 1), jnp.float32)], axis=1)
    b2p = jnp.concatenate(
        [b2[1:], b2[0:1], jnp.zeros((_D - 1,), jnp.float32)])[None, :]
    b1r = b1[None, :]
    ber = be[None, :]
    brr = br[None, :]
    We1 = We[:4 * _D]
    We2 = We[4 * _D:]

    a = in_vc[src]
    b = in_vc[dst]

    f, vepre, est = pl.pallas_call(
        _edge_kernel,
        grid=(_E // _EBLK,),
        in_specs=[
            pl.BlockSpec((_EBLK, _D), lambda i: (i, 0)),
            pl.BlockSpec((_EBLK, _D), lambda i: (i, 0)),
            pl.BlockSpec((_EBLK, _D), lambda i: (i, 0)),
            pl.BlockSpec((3 * _D, 3 * _D), lambda i: (0, 0)),
            pl.BlockSpec((1, 3 * _D), lambda i: (0, 0)),
            pl.BlockSpec((3 * _D, 5 * _D), lambda i: (0, 0)),
            pl.BlockSpec((1, 5 * _D), lambda i: (0, 0)),
            pl.BlockSpec((4 * _D, _D), lambda i: (0, 0)),
            pl.BlockSpec((_D, _D), lambda i: (0, 0)),
            pl.BlockSpec((1, _D), lambda i: (0, 0)),
        ],
        out_specs=[
            pl.BlockSpec((_EBLK, 4 * _D), lambda i: (i, 0)),
            pl.BlockSpec((_EBLK, _D), lambda i: (i, 0)),
            pl.BlockSpec((8, _D), lambda i: (0, 0)),
        ],
        out_shape=[
            jax.ShapeDtypeStruct((_E, 4 * _D), jnp.float32),
            jax.ShapeDtypeStruct((_E, _D), jnp.float32),
            jax.ShapeDtypeStruct((8, _D), jnp.float32),
        ],
    )(a, b, in_ve, W1, b1r, W2p, b2p, We1, We2, ber)

    f1 = f[:, :_D]
    f2 = f[:, _D:2 * _D]
    f3 = f[:, 2 * _D:3 * _D]
    f4 = f[:, 3 * _D:]

    deg = jax.ops.segment_sum(jnp.ones((_E,), jnp.float32), dst,
                              num_segments=_N)
    has = (deg > 0)[:, None]
    nf1 = jax.ops.segment_sum(f1, dst, num_segments=_N)
    nf2 = jnp.where(has, jax.ops.segment_max(f2, dst, num_segments=_N), 0.0)
    nf3 = jnp.where(has, jax.ops.segment_min(f3, dst, num_segments=_N), 0.0)
    nf4 = jax.ops.segment_sum(f4, dst, num_segments=_N)
    nf4 = nf4 / jnp.maximum(deg, 1.0)[:, None]

    vcpre, nst = pl.pallas_call(
        _node_kernel,
        grid=(_N // _NBLK,),
        in_specs=[
            pl.BlockSpec((_NBLK, _D), lambda i: (i, 0)),
            pl.BlockSpec((_NBLK, _D), lambda i: (i, 0)),
            pl.BlockSpec((_NBLK, _D), lambda i: (i, 0)),
            pl.BlockSpec((_NBLK, _D), lambda i: (i, 0)),
            pl.BlockSpec((_NBLK, _D), lambda i: (i, 0)),
            pl.BlockSpec((5 * _D, _D), lambda i: (0, 0)),
            pl.BlockSpec((1, _D), lambda i: (0, 0)),
        ],
        out_specs=[
            pl.BlockSpec((_NBLK, _D), lambda i: (i, 0)),
            pl.BlockSpec((8, _D), lambda i: (0, 0)),
        ],
        out_shape=[
            jax.ShapeDtypeStruct((_N, _D), jnp.float32),
            jax.ShapeDtypeStruct((8, _D), jnp.float32),
        ],
    )(in_vc, nf1, nf2, nf3, nf4, Wr, brr)

    def _bn_coeffs(stats, n, gamma, beta):
        mu = stats[0] / n
        var = stats[1] / n - mu * mu
        rstd = jax.lax.rsqrt(var + _EPS)
        scale = gamma * rstd
        shift = beta - mu * scale
        return scale[None, :], shift[None, :]

    esc, esh = _bn_coeffs(est, float(_E), gamma_ef, beta_ef)
    nsc, nsh = _bn_coeffs(nst, float(_N), gamma_gc, beta_gc)

    out_vc = _normalize(vcpre, nsc, nsh, _NBLK)
    out_ve = _normalize(vepre, esc, esh, 2000)
    return (out_vc, out_ve)


# speedup vs baseline: 1.3899x; 1.3899x over previous
import functools
import jax
import jax.numpy as jnp
from jax import lax
from jax.experimental import pallas as pl
from jax.experimental.pallas import tpu as pltpu
from jax.experimental.pallas import tpu_sc as plsc

_D = 128
_N = 10000
_E = 320000
_EPS = 1e-5
_EBLK = 512
_NBLK = 1000

_NW = 32
_GCHUNK = 400


def _sc_gather2(table, src, dst):
    per_w = _E // _NW
    n_it = per_w // _GCHUNK
    mesh = plsc.VectorSubcoreMesh(core_axis_name="c", subcore_axis_name="s")

    @functools.partial(
        pl.kernel,
        out_type=[jax.ShapeDtypeStruct((_E, _D), jnp.float32),
                  jax.ShapeDtypeStruct((_E, _D), jnp.float32)],
        mesh=mesh,
        scratch_types=[
            pltpu.VMEM((_GCHUNK,), jnp.int32),
            pltpu.VMEM((_GCHUNK, _D), jnp.float32),
            pltpu.SemaphoreType.DMA,
        ],
    )
    def k(table_hbm, src_hbm, dst_hbm, a_hbm, b_hbm, idx_v, rows_v, sem):
        wid = lax.axis_index("s") * 2 + lax.axis_index("c")
        base0 = wid * per_w

        def step(idx_hbm, out_hbm, j):
            base = base0 + j * _GCHUNK
            pltpu.sync_copy(idx_hbm.at[pl.ds(base, _GCHUNK)], idx_v)
            pltpu.async_copy(table_hbm.at[idx_v], rows_v, sem).wait()
            pltpu.sync_copy(rows_v, out_hbm.at[pl.ds(base, _GCHUNK)])

        def body(j, carry):
            step(src_hbm, a_hbm, j)
            step(dst_hbm, b_hbm, j)
            return carry

        lax.fori_loop(0, n_it, body, 0)

    return k(table, src, dst)


def _edge_kernel(a_ref, b_ref, ve_ref, W1_ref, b1_ref, W2p_ref, b2p_ref,
                 We1_ref, We2_ref, be_ref, f1_ref, f2_ref, f3_ref, f4_ref,
                 vepre_ref, stats_ref):
    x = jnp.concatenate([a_ref[...], b_ref[...], ve_ref[...]], axis=1)
    h = jnp.maximum(x @ W1_ref[...] + b1_ref[...], 0.0)
    m = h @ W2p_ref[...] + b2p_ref[...]
    k = jax.nn.sigmoid(m[:, 4 * _D:4 * _D + 1])
    f = m[:, :4 * _D] * k
    f1_ref[...] = f[:, :_D]
    f2_ref[...] = f[:, _D:2 * _D]
    f3_ref[...] = f[:, 2 * _D:3 * _D]
    f4_ref[...] = f[:, 3 * _D:]
    vp = f @ We1_ref[...] + ve_ref[...] @ We2_ref[...] + be_ref[...]
    vepre_ref[...] = vp

    @pl.when(pl.program_id(0) == 0)
    def _():
        stats_ref[...] = jnp.zeros_like(stats_ref)

    s = jnp.sum(vp, axis=0, keepdims=True)
    s2 = jnp.sum(vp * vp, axis=0, keepdims=True)
    stats_ref[0:1, :] += s
    stats_ref[1:2, :] += s2


def _node_kernel(vc_ref, nf1_ref, nf2_ref, nf3_ref, nf4_ref, Wr_ref, br_ref,
                 out_ref, stats_ref):
    Wr = Wr_ref[...]
    o = (vc_ref[...] @ Wr[0:_D] + nf1_ref[...] @ Wr[_D:2 * _D]
         + nf2_ref[...] @ Wr[2 * _D:3 * _D] + nf3_ref[...] @ Wr[3 * _D:4 * _D]
         + nf4_ref[...] @ Wr[4 * _D:5 * _D] + br_ref[...])
    out_ref[...] = o

    @pl.when(pl.program_id(0) == 0)
    def _():
        stats_ref[...] = jnp.zeros_like(stats_ref)

    stats_ref[0:1, :] += jnp.sum(o, axis=0, keepdims=True)
    stats_ref[1:2, :] += jnp.sum(o * o, axis=0, keepdims=True)


def _norm_kernel(x_ref, scale_ref, shift_ref, o_ref):
    o_ref[...] = x_ref[...] * scale_ref[...] + shift_ref[...]


def _normalize(x, scale, shift, blk):
    n = x.shape[0]
    return pl.pallas_call(
        _norm_kernel,
        grid=(n // blk,),
        in_specs=[
            pl.BlockSpec((blk, _D), lambda i: (i, 0)),
            pl.BlockSpec((1, _D), lambda i: (0, 0)),
            pl.BlockSpec((1, _D), lambda i: (0, 0)),
        ],
        out_specs=pl.BlockSpec((blk, _D), lambda i: (i, 0)),
        out_shape=jax.ShapeDtypeStruct((n, _D), jnp.float32),
    )(x, scale, shift)


def kernel(in_vc, in_ve, edge_index, W1, b1, W2, b2, Wr, br, We, be,
           gamma_gc, beta_gc, gamma_ef, beta_ef):
    src = edge_index[0]
    dst = edge_index[1]

    W2p = jnp.concatenate(
        [W2[:, 1:], W2[:, 0:1], jnp.zeros((3 * _D, _D - 1), jnp.float32)], axis=1)
    b2p = jnp.concatenate(
        [b2[1:], b2[0:1], jnp.zeros((_D - 1,), jnp.float32)])[None, :]
    b1r = b1[None, :]
    ber = be[None, :]
    brr = br[None, :]
    We1 = We[:4 * _D]
    We2 = We[4 * _D:]

    a, b = _sc_gather2(in_vc, src, dst)

    f1, f2, f3, f4, vepre, est = pl.pallas_call(
        _edge_kernel,
        grid=(_E // _EBLK,),
        in_specs=[
            pl.BlockSpec((_EBLK, _D), lambda i: (i, 0)),
            pl.BlockSpec((_EBLK, _D), lambda i: (i, 0)),
            pl.BlockSpec((_EBLK, _D), lambda i: (i, 0)),
            pl.BlockSpec((3 * _D, 3 * _D), lambda i: (0, 0)),
            pl.BlockSpec((1, 3 * _D), lambda i: (0, 0)),
            pl.BlockSpec((3 * _D, 5 * _D), lambda i: (0, 0)),
            pl.BlockSpec((1, 5 * _D), lambda i: (0, 0)),
            pl.BlockSpec((4 * _D, _D), lambda i: (0, 0)),
            pl.BlockSpec((_D, _D), lambda i: (0, 0)),
            pl.BlockSpec((1, _D), lambda i: (0, 0)),
        ],
        out_specs=[
            pl.BlockSpec((_EBLK, _D), lambda i: (i, 0)),
            pl.BlockSpec((_EBLK, _D), lambda i: (i, 0)),
            pl.BlockSpec((_EBLK, _D), lambda i: (i, 0)),
            pl.BlockSpec((_EBLK, _D), lambda i: (i, 0)),
            pl.BlockSpec((_EBLK, _D), lambda i: (i, 0)),
            pl.BlockSpec((8, _D), lambda i: (0, 0)),
        ],
        out_shape=[
            jax.ShapeDtypeStruct((_E, _D), jnp.float32),
            jax.ShapeDtypeStruct((_E, _D), jnp.float32),
            jax.ShapeDtypeStruct((_E, _D), jnp.float32),
            jax.ShapeDtypeStruct((_E, _D), jnp.float32),
            jax.ShapeDtypeStruct((_E, _D), jnp.float32),
            jax.ShapeDtypeStruct((8, _D), jnp.float32),
        ],
    )(a, b, in_ve, W1, b1r, W2p, b2p, We1, We2, ber)

    deg = jax.ops.segment_sum(jnp.ones((_E,), jnp.float32), dst,
                              num_segments=_N)
    has = (deg > 0)[:, None]
    nf1 = jax.ops.segment_sum(f1, dst, num_segments=_N)
    nf2 = jnp.where(has, jax.ops.segment_max(f2, dst, num_segments=_N), 0.0)
    nf3 = jnp.where(has, jax.ops.segment_min(f3, dst, num_segments=_N), 0.0)
    nf4 = jax.ops.segment_sum(f4, dst, num_segments=_N)
    nf4 = nf4 / jnp.maximum(deg, 1.0)[:, None]

    vcpre, nst = pl.pallas_call(
        _node_kernel,
        grid=(_N // _NBLK,),
        in_specs=[
            pl.BlockSpec((_NBLK, _D), lambda i: (i, 0)),
            pl.BlockSpec((_NBLK, _D), lambda i: (i, 0)),
            pl.BlockSpec((_NBLK, _D), lambda i: (i, 0)),
            pl.BlockSpec((_NBLK, _D), lambda i: (i, 0)),
            pl.BlockSpec((_NBLK, _D), lambda i: (i, 0)),
            pl.BlockSpec((5 * _D, _D), lambda i: (0, 0)),
            pl.BlockSpec((1, _D), lambda i: (0, 0)),
        ],
        out_specs=[
            pl.BlockSpec((_NBLK, _D), lambda i: (i, 0)),
            pl.BlockSpec((8, _D), lambda i: (0, 0)),
        ],
        out_shape=[
            jax.ShapeDtypeStruct((_N, _D), jnp.float32),
            jax.ShapeDtypeStruct((8, _D), jnp.float32),
        ],
    )(in_vc, nf1, nf2, nf3, nf4, Wr, brr)

    def _bn_coeffs(stats, n, gamma, beta):
        mu = stats[0] / n
        var = stats[1] / n - mu * mu
        rstd = jax.lax.rsqrt(var + _EPS)
        scale = gamma * rstd
        shift = beta - mu * scale
        return scale[None, :], shift[None, :]

    esc, esh = _bn_coeffs(est, float(_E), gamma_ef, beta_ef)
    nsc, nsh = _bn_coeffs(nst, float(_N), gamma_gc, beta_gc)

    out_vc = _normalize(vcpre, nsc, nsh, _NBLK)
    out_ve = _normalize(vepre, esc, esh, 2000)
    return (out_vc, out_ve)


# speedup vs baseline: 1.4734x; 1.0601x over previous
import functools
import jax
import jax.numpy as jnp
from jax import lax
from jax.experimental import pallas as pl
from jax.experimental.pallas import tpu as pltpu
from jax.experimental.pallas import tpu_sc as plsc

_D = 128
_N = 10000
_E = 320000
_EPS = 1e-5
_EBLK = 640
_NBLK = 1000

_NW = 32
_GCHUNK = 400


def _sc_gather2(table, src, dst):
    per_w = _E // _NW
    n_it = per_w // _GCHUNK
    mesh = plsc.VectorSubcoreMesh(core_axis_name="c", subcore_axis_name="s")

    @functools.partial(
        pl.kernel,
        out_type=[jax.ShapeDtypeStruct((_E, _D), jnp.float32),
                  jax.ShapeDtypeStruct((_E, _D), jnp.float32)],
        mesh=mesh,
        scratch_types=[
            pltpu.VMEM((_GCHUNK,), jnp.int32),
            pltpu.VMEM((_GCHUNK, _D), jnp.float32),
            pltpu.SemaphoreType.DMA,
        ],
    )
    def k(table_hbm, src_hbm, dst_hbm, a_hbm, b_hbm, idx_v, rows_v, sem):
        wid = lax.axis_index("s") * 2 + lax.axis_index("c")
        base0 = wid * per_w

        def step(idx_hbm, out_hbm, j):
            base = base0 + j * _GCHUNK
            pltpu.sync_copy(idx_hbm.at[pl.ds(base, _GCHUNK)], idx_v)
            pltpu.async_copy(table_hbm.at[idx_v], rows_v, sem).wait()
            pltpu.sync_copy(rows_v, out_hbm.at[pl.ds(base, _GCHUNK)])

        def body(j, carry):
            step(src_hbm, a_hbm, j)
            step(dst_hbm, b_hbm, j)
            return carry

        lax.fori_loop(0, n_it, body, 0)

    return k(table, src, dst)


def _edge_kernel(a_ref, b_ref, ve_ref, W1_ref, b1_ref, W2p_ref, b2p_ref,
                 We1_ref, We2_ref, be_ref, f1_ref, f2_ref, f3_ref, f4_ref,
                 vepre_ref, stats_ref):
    x = jnp.concatenate([a_ref[...], b_ref[...], ve_ref[...]], axis=1)
    h = jnp.maximum(x @ W1_ref[...] + b1_ref[...], 0.0)
    m = h @ W2p_ref[...] + b2p_ref[...]
    k = jax.nn.sigmoid(m[:, 4 * _D:4 * _D + 1])
    f = m[:, :4 * _D] * k
    f1_ref[...] = f[:, :_D]
    f2_ref[...] = f[:, _D:2 * _D]
    f3_ref[...] = f[:, 2 * _D:3 * _D]
    f4_ref[...] = f[:, 3 * _D:]
    vp = f @ We1_ref[...] + ve_ref[...] @ We2_ref[...] + be_ref[...]
    vepre_ref[...] = vp

    @pl.when(pl.program_id(0) == 0)
    def _():
        stats_ref[...] = jnp.zeros_like(stats_ref)

    s = jnp.sum(vp, axis=0, keepdims=True)
    s2 = jnp.sum(vp * vp, axis=0, keepdims=True)
    stats_ref[0:1, :] += s
    stats_ref[1:2, :] += s2


def _node_kernel(vc_ref, nf1_ref, nf2_ref, nf3_ref, nf4_ref, Wr_ref, br_ref,
                 out_ref, stats_ref):
    Wr = Wr_ref[...]
    o = (vc_ref[...] @ Wr[0:_D] + nf1_ref[...] @ Wr[_D:2 * _D]
         + nf2_ref[...] @ Wr[2 * _D:3 * _D] + nf3_ref[...] @ Wr[3 * _D:4 * _D]
         + nf4_ref[...] @ Wr[4 * _D:5 * _D] + br_ref[...])
    out_ref[...] = o

    @pl.when(pl.program_id(0) == 0)
    def _():
        stats_ref[...] = jnp.zeros_like(stats_ref)

    stats_ref[0:1, :] += jnp.sum(o, axis=0, keepdims=True)
    stats_ref[1:2, :] += jnp.sum(o * o, axis=0, keepdims=True)


def _norm_kernel(x_ref, scale_ref, shift_ref, o_ref):
    o_ref[...] = x_ref[...] * scale_ref[...] + shift_ref[...]


def _normalize(x, scale, shift, blk):
    n = x.shape[0]
    return pl.pallas_call(
        _norm_kernel,
        grid=(n // blk,),
        in_specs=[
            pl.BlockSpec((blk, _D), lambda i: (i, 0)),
            pl.BlockSpec((1, _D), lambda i: (0, 0)),
            pl.BlockSpec((1, _D), lambda i: (0, 0)),
        ],
        out_specs=pl.BlockSpec((blk, _D), lambda i: (i, 0)),
        out_shape=jax.ShapeDtypeStruct((n, _D), jnp.float32),
    )(x, scale, shift)


def kernel(in_vc, in_ve, edge_index, W1, b1, W2, b2, Wr, br, We, be,
           gamma_gc, beta_gc, gamma_ef, beta_ef):
    src = edge_index[0]
    dst = edge_index[1]

    W2p = jnp.concatenate(
        [W2[:, 1:], W2[:, 0:1], jnp.zeros((3 * _D, _D - 1), jnp.float32)], axis=1)
    b2p = jnp.concatenate(
        [b2[1:], b2[0:1], jnp.zeros((_D - 1,), jnp.float32)])[None, :]
    b1r = b1[None, :]
    ber = be[None, :]
    brr = br[None, :]
    We1 = We[:4 * _D]
    We2 = We[4 * _D:]

    a, b = _sc_gather2(in_vc, src, dst)

    deg = jax.ops.segment_sum(jnp.ones((_E,), jnp.float32), dst,
                              num_segments=_N)

    half = _E // 2
    nhb = half // _EBLK

    def _edge_half(off):
        return pl.pallas_call(
            _edge_kernel,
            grid=(nhb,),
            in_specs=[
                pl.BlockSpec((_EBLK, _D), lambda i: (i + off, 0)),
                pl.BlockSpec((_EBLK, _D), lambda i: (i + off, 0)),
                pl.BlockSpec((_EBLK, _D), lambda i: (i + off, 0)),
                pl.BlockSpec((3 * _D, 3 * _D), lambda i: (0, 0)),
                pl.BlockSpec((1, 3 * _D), lambda i: (0, 0)),
                pl.BlockSpec((3 * _D, 5 * _D), lambda i: (0, 0)),
                pl.BlockSpec((1, 5 * _D), lambda i: (0, 0)),
                pl.BlockSpec((4 * _D, _D), lambda i: (0, 0)),
                pl.BlockSpec((_D, _D), lambda i: (0, 0)),
                pl.BlockSpec((1, _D), lambda i: (0, 0)),
            ],
            out_specs=[
                pl.BlockSpec((_EBLK, _D), lambda i: (i, 0)),
                pl.BlockSpec((_EBLK, _D), lambda i: (i, 0)),
                pl.BlockSpec((_EBLK, _D), lambda i: (i, 0)),
                pl.BlockSpec((_EBLK, _D), lambda i: (i, 0)),
                pl.BlockSpec((_EBLK, _D), lambda i: (i, 0)),
                pl.BlockSpec((8, _D), lambda i: (0, 0)),
            ],
            out_shape=[
                jax.ShapeDtypeStruct((half, _D), jnp.float32),
                jax.ShapeDtypeStruct((half, _D), jnp.float32),
                jax.ShapeDtypeStruct((half, _D), jnp.float32),
                jax.ShapeDtypeStruct((half, _D), jnp.float32),
                jax.ShapeDtypeStruct((half, _D), jnp.float32),
                jax.ShapeDtypeStruct((8, _D), jnp.float32),
            ],
        )(a, b, in_ve, W1, b1r, W2p, b2p, We1, We2, ber)

    dst1 = dst[:half]
    dst2 = dst[half:]
    f1a, f2a, f3a, f4a, vpa, esta = _edge_half(0)
    s1a = jax.ops.segment_sum(f1a, dst1, num_segments=_N)
    m2a = jax.ops.segment_max(f2a, dst1, num_segments=_N)
    m3a = jax.ops.segment_min(f3a, dst1, num_segments=_N)
    s4a = jax.ops.segment_sum(f4a, dst1, num_segments=_N)
    f1b, f2b, f3b, f4b, vpb, estb = _edge_half(nhb)
    s1b = jax.ops.segment_sum(f1b, dst2, num_segments=_N)
    m2b = jax.ops.segment_max(f2b, dst2, num_segments=_N)
    m3b = jax.ops.segment_min(f3b, dst2, num_segments=_N)
    s4b = jax.ops.segment_sum(f4b, dst2, num_segments=_N)

    vepre = jnp.concatenate([vpa, vpb], axis=0)
    est = esta + estb

    has = (deg > 0)[:, None]
    nf1 = s1a + s1b
    nf2 = jnp.where(has, jnp.maximum(m2a, m2b), 0.0)
    nf3 = jnp.where(has, jnp.minimum(m3a, m3b), 0.0)
    nf4 = (s4a + s4b) / jnp.maximum(deg, 1.0)[:, None]

    vcpre, nst = pl.pallas_call(
        _node_kernel,
        grid=(_N // _NBLK,),
        in_specs=[
            pl.BlockSpec((_NBLK, _D), lambda i: (i, 0)),
            pl.BlockSpec((_NBLK, _D), lambda i: (i, 0)),
            pl.BlockSpec((_NBLK, _D), lambda i: (i, 0)),
            pl.BlockSpec((_NBLK, _D), lambda i: (i, 0)),
            pl.BlockSpec((_NBLK, _D), lambda i: (i, 0)),
            pl.BlockSpec((5 * _D, _D), lambda i: (0, 0)),
            pl.BlockSpec((1, _D), lambda i: (0, 0)),
        ],
        out_specs=[
            pl.BlockSpec((_NBLK, _D), lambda i: (i, 0)),
            pl.BlockSpec((8, _D), lambda i: (0, 0)),
        ],
        out_shape=[
            jax.ShapeDtypeStruct((_N, _D), jnp.float32),
            jax.ShapeDtypeStruct((8, _D), jnp.float32),
        ],
    )(in_vc, nf1, nf2, nf3, nf4, Wr, brr)

    def _bn_coeffs(stats, n, gamma, beta):
        mu = stats[0] / n
        var = stats[1] / n - mu * mu
        rstd = jax.lax.rsqrt(var + _EPS)
        scale = gamma * rstd
        shift = beta - mu * scale
        return scale[None, :], shift[None, :]

    esc, esh = _bn_coeffs(est, float(_E), gamma_ef, beta_ef)
    nsc, nsh = _bn_coeffs(nst, float(_N), gamma_gc, beta_gc)

    out_vc = _normalize(vcpre, nsc, nsh, _NBLK)
    out_ve = _normalize(vepre, esc, esh, 2000)
    return (out_vc, out_ve)
